# baseline (speedup 1.0000x reference)
import jax
import jax.numpy as jnp
from jax import lax
from jax.experimental import pallas as pl
from jax.experimental.pallas import tpu as pltpu

def _w1_chunks(H):
    sizes = (H // 8, H // 4, H // 4, 3 * H // 8)
    out, base = [], 0
    for s in sizes:
        out.append((base, s))
        base += s
    return tuple(out)


_NCHUNKS2 = 2


def _dot_t(a, b):
    return lax.dot_general(
        a, b,
        dimension_numbers=(((1,), (1,)), ((), ())),
        preferred_element_type=jnp.float32,
    )


def _mlp_kernel(x_ref, w1_hbm, b1_ref, w2_hbm, b2_ref, o_ref,
                w1_ref, w2_ref, h_ref, sems):
    i = pl.program_id(0)
    O = w2_ref.shape[0]
    OC = O // _NCHUNKS2
    w1_chunks = _w1_chunks(w1_ref.shape[0])

    def w1_chunk_copy(c):
        base, rows = w1_chunks[c]
        return pltpu.make_async_copy(
            w1_hbm.at[pl.ds(base, rows), :], w1_ref.at[pl.ds(base, rows), :],
            sems.at[c])

    def w2_chunk_copy(c):
        return pltpu.make_async_copy(
            w2_hbm.at[pl.ds(c * OC, OC), :], w2_ref.at[pl.ds(c * OC, OC), :],
            sems.at[len(w1_chunks) + c])

    @pl.when(i == 0)
    def _first_step():
        for c in range(len(w1_chunks)):
            w1_chunk_copy(c).start()
        for c in range(_NCHUNKS2):
            w2_chunk_copy(c).start()
        for c, (base, rows) in enumerate(w1_chunks):
            w1_chunk_copy(c).wait()
            h = _dot_t(x_ref[...], w1_ref[pl.ds(base, rows), :])
            h_ref[:, pl.ds(base, rows)] = jnp.maximum(
                h + b1_ref[:, pl.ds(base, rows)], 0.0)
        for c in range(_NCHUNKS2):
            w2_chunk_copy(c).wait()
            y = _dot_t(h_ref[...], w2_ref[pl.ds(c * OC, OC), :])
            o_ref[:, pl.ds(c * OC, OC)] = y + b2_ref[:, pl.ds(c * OC, OC)]

    @pl.when(i != 0)
    def _steady_state():
        h = _dot_t(x_ref[...], w1_ref[...])
        h_ref[...] = jnp.maximum(h + b1_ref[...], 0.0)
        y = _dot_t(h_ref[...], w2_ref[...])
        o_ref[...] = y + b2_ref[...]


def kernel(x, w1, b1, w2, b2):
    B, Din = x.shape
    H = w1.shape[0]
    O = w2.shape[0]

    TB = 512
    B_pad = ((B + TB - 1) // TB) * TB
    xp = jnp.pad(x, ((0, B_pad - B), (0, 0))) if B_pad != B else x
    b1_2d = b1.reshape(1, H)
    b2_2d = b2.reshape(1, O)

    out = pl.pallas_call(
        _mlp_kernel,
        out_shape=jax.ShapeDtypeStruct((B_pad, O), jnp.float32),
        grid=(B_pad // TB,),
        in_specs=[
            pl.BlockSpec((TB, Din), lambda i: (i, 0)),
            pl.BlockSpec(memory_space=pl.ANY),
            pl.BlockSpec((1, H), lambda i: (0, 0)),
            pl.BlockSpec(memory_space=pl.ANY),
            pl.BlockSpec((1, O), lambda i: (0, 0)),
        ],
        out_specs=pl.BlockSpec((TB, O), lambda i: (i, 0)),
        scratch_shapes=[
            pltpu.VMEM((H, Din), jnp.float32),
            pltpu.VMEM((O, H), jnp.float32),
            pltpu.VMEM((TB, H), jnp.float32),
            pltpu.SemaphoreType.DMA((4 + _NCHUNKS2,)),
        ],
        compiler_params=pltpu.CompilerParams(
            dimension_semantics=("arbitrary",),
        ),
    )(xp, w1, b1_2d, w2, b2_2d)
    return out[:B] if B_pad != B else out
